# baseline (speedup 1.0000x reference)
import functools

import jax
import jax.numpy as jnp
from jax.experimental import pallas as pl
from jax.experimental.pallas import tpu as pltpu

_EPAD = 128


def _body(E, sp_ref, x_ref, w1_ref, b1_ref, w2p_ref, b2p_ref, tab_ref,
          out_ref, log_ref):
    j = pl.program_id(0)
    nj = pl.num_programs(0)

    pre = jnp.dot(x_ref[...], w1_ref[...], preferred_element_type=jnp.float32)
    h = jax.nn.gelu(pre + b1_ref[...])
    plog = jnp.dot(h.astype(jnp.bfloat16), w2p_ref[...],
                   preferred_element_type=jnp.float32)

    @pl.when(j == 0)
    def _():
        log_ref[...] = plog + b2p_ref[...]

    @pl.when(j != 0)
    def _():
        log_ref[...] = log_ref[...] + plog

    @pl.when(j == nj - 1)
    def _():
        lg = log_ref[...]
        m = jnp.max(lg)
        rows = jax.lax.broadcasted_iota(jnp.int32, lg.shape, 0)
        cols = jax.lax.broadcasted_iota(jnp.int32, lg.shape, 1)
        flat = rows * E + cols
        idx = jnp.min(jnp.where(lg == m, flat, jnp.int32(2**30)))
        row2 = tab_ref[0, pl.ds(idx // 2, 1), :]
        half = row2.shape[-1] // 2
        out_ref[...] = jnp.where((idx % 2) == 0, row2[:, :half], row2[:, half:])


def kernel(predicate, W1, b1, W2, b2, expert_tables, input):
    T, D = predicate.shape
    H = W1.shape[1]
    E = W2.shape[1]
    n_tab, ROWS, ED = expert_tables.shape
    tab2 = expert_tables.reshape(n_tab, ROWS // 2, 2 * ED)

    HB = 512
    J = H // HB

    xb = predicate.astype(jnp.bfloat16)
    W1b = W1.astype(jnp.bfloat16)
    W2p = jnp.zeros((H, _EPAD), jnp.bfloat16).at[:, :E].set(
        W2.astype(jnp.bfloat16))
    b2p = jnp.full((1, _EPAD), -1e30, jnp.float32).at[0, :E].set(b2)
    b1r = b1.reshape(1, H)
    sp = jnp.asarray(input, jnp.int32).reshape(1)

    grid_spec = pltpu.PrefetchScalarGridSpec(
        num_scalar_prefetch=1,
        grid=(J,),
        in_specs=[
            pl.BlockSpec((T, D), lambda j, sp: (0, 0)),
            pl.BlockSpec((D, HB), lambda j, sp: (0, j)),
            pl.BlockSpec((1, HB), lambda j, sp: (0, j)),
            pl.BlockSpec((HB, _EPAD), lambda j, sp: (j, 0)),
            pl.BlockSpec((1, _EPAD), lambda j, sp: (0, 0)),
            pl.BlockSpec((1, ROWS // 2, 2 * ED), lambda j, sp: (sp[0], 0, 0)),
        ],
        out_specs=pl.BlockSpec((1, ED), lambda j, sp: (0, 0)),
        scratch_shapes=[
            pltpu.VMEM((T, _EPAD), jnp.float32),
        ],
    )

    out = pl.pallas_call(
        functools.partial(_body, E),
        grid_spec=grid_spec,
        out_shape=jax.ShapeDtypeStruct((1, ED), jnp.float32),
        compiler_params=pltpu.CompilerParams(
            dimension_semantics=("arbitrary",),
        ),
    )(sp, xb, W1b, b1r, W2p, b2p, tab2)
    return out.reshape(ED)


# speedup vs baseline: 1.1952x; 1.1952x over previous
import functools

import jax
import jax.numpy as jnp
from jax.experimental import pallas as pl
from jax.experimental.pallas import tpu as pltpu

_EPAD = 128


def _body(E, S, DB, HB, sp_ref, xc_ref, w1_ref, b1_ref, w2p_ref, b2p_ref,
          tab_ref, out_ref, xbf_ref, log_ref):
    s = pl.program_id(0)
    ns = pl.num_programs(0)

    @pl.when(s < S)
    def _():
        xbf_ref[s] = xc_ref[...].astype(jnp.bfloat16)

    @pl.when(s >= S)
    def _():
        j = s - S
        w1b = w1_ref[...].astype(jnp.bfloat16)
        pre = jnp.zeros((xbf_ref.shape[1], HB), jnp.float32)
        for k in range(S):
            pre = pre + jnp.dot(xbf_ref[k],
                                w1b[k * DB:(k + 1) * DB, :],
                                preferred_element_type=jnp.float32)
        h = jax.nn.gelu(pre + b1_ref[...])
        plog = jnp.dot(h.astype(jnp.bfloat16), w2p_ref[...],
                       preferred_element_type=jnp.float32)

        @pl.when(j == 0)
        def _():
            log_ref[...] = plog + b2p_ref[...]

        @pl.when(j != 0)
        def _():
            log_ref[...] = log_ref[...] + plog

        @pl.when(s == ns - 1)
        def _():
            lg = log_ref[...]
            m = jnp.max(lg)
            rows = jax.lax.broadcasted_iota(jnp.int32, lg.shape, 0)
            cols = jax.lax.broadcasted_iota(jnp.int32, lg.shape, 1)
            flat = rows * E + cols
            idx = jnp.min(jnp.where(lg == m, flat, jnp.int32(2**30)))
            row2 = tab_ref[0, pl.ds(idx // 2, 1), :]
            half = row2.shape[-1] // 2
            out_ref[...] = jnp.where((idx % 2) == 0,
                                     row2[:, :half], row2[:, half:])


def kernel(predicate, W1, b1, W2, b2, expert_tables, input):
    T, D = predicate.shape
    H = W1.shape[1]
    E = W2.shape[1]
    n_tab, ROWS, ED = expert_tables.shape
    tab2 = expert_tables.reshape(n_tab, ROWS // 2, 2 * ED)

    DB = 1024
    S = D // DB
    HB = 256
    J = H // HB

    W2p = jnp.zeros((H, _EPAD), jnp.float32).at[:, :E].set(W2)
    W2pb = W2p.astype(jnp.bfloat16)
    b2p = jnp.full((1, _EPAD), -1e30, jnp.float32).at[0, :E].set(b2)
    b1r = b1.reshape(1, H)
    sp = jnp.asarray(input, jnp.int32).reshape(1)

    grid_spec = pltpu.PrefetchScalarGridSpec(
        num_scalar_prefetch=1,
        grid=(S + J,),
        in_specs=[
            pl.BlockSpec((T, DB), lambda s, sp: (0, jnp.minimum(s, S - 1))),
            pl.BlockSpec((D, HB),
                         lambda s, sp: (0, jnp.clip(s - S, 0, J - 1))),
            pl.BlockSpec((1, HB),
                         lambda s, sp: (0, jnp.clip(s - S, 0, J - 1))),
            pl.BlockSpec((HB, _EPAD),
                         lambda s, sp: (jnp.clip(s - S, 0, J - 1), 0)),
            pl.BlockSpec((1, _EPAD), lambda s, sp: (0, 0)),
            pl.BlockSpec((1, ROWS // 2, 2 * ED), lambda s, sp: (sp[0], 0, 0)),
        ],
        out_specs=pl.BlockSpec((1, ED), lambda s, sp: (0, 0)),
        scratch_shapes=[
            pltpu.VMEM((S, T, DB), jnp.bfloat16),
            pltpu.VMEM((T, _EPAD), jnp.float32),
        ],
    )

    out = pl.pallas_call(
        functools.partial(_body, E, S, DB, HB),
        grid_spec=grid_spec,
        out_shape=jax.ShapeDtypeStruct((1, ED), jnp.float32),
        compiler_params=pltpu.CompilerParams(
            dimension_semantics=("arbitrary",),
        ),
    )(sp, predicate, W1, b1r, W2pb, b2p, tab2)
    return out.reshape(ED)


# speedup vs baseline: 1.5716x; 1.3149x over previous
import functools

import jax
import jax.numpy as jnp
from jax.experimental import pallas as pl
from jax.experimental.pallas import tpu as pltpu

_EPAD = 128


def _body(E, S, DB, HB, sp_ref, xc_ref, w1_ref, b1_ref, w2p_ref, b2p_ref,
          tab_ref, out_ref, xbf_ref, log_ref, row_ref, sem):
    s = pl.program_id(0)
    ns = pl.num_programs(0)

    @pl.when(s < S)
    def _():
        xbf_ref[s] = xc_ref[...].astype(jnp.bfloat16)

    @pl.when(s >= S)
    def _():
        j = s - S
        w1b = w1_ref[...].astype(jnp.bfloat16)
        pre = jnp.zeros((xbf_ref.shape[1], HB), jnp.float32)
        for k in range(S):
            pre = pre + jnp.dot(xbf_ref[k],
                                w1b[k * DB:(k + 1) * DB, :],
                                preferred_element_type=jnp.float32)
        h = jax.nn.gelu(pre + b1_ref[...])
        plog = jnp.dot(h.astype(jnp.bfloat16), w2p_ref[...],
                       preferred_element_type=jnp.float32)

        @pl.when(j == 0)
        def _():
            log_ref[...] = plog + b2p_ref[...]

        @pl.when(j != 0)
        def _():
            log_ref[...] = log_ref[...] + plog

        @pl.when(s == ns - 1)
        def _():
            lg = log_ref[...]
            m = jnp.max(lg)
            rows = jax.lax.broadcasted_iota(jnp.int32, lg.shape, 0)
            cols = jax.lax.broadcasted_iota(jnp.int32, lg.shape, 1)
            flat = rows * E + cols
            idx = jnp.min(jnp.where(lg == m, flat, jnp.int32(2**30)))
            copy = pltpu.make_async_copy(
                tab_ref.at[sp_ref[0], pl.ds(idx, 1), :], row_ref, sem)
            copy.start()
            copy.wait()
            out_ref[...] = row_ref[...]


def kernel(predicate, W1, b1, W2, b2, expert_tables, input):
    T, D = predicate.shape
    H = W1.shape[1]
    E = W2.shape[1]
    n_tab, ROWS, ED = expert_tables.shape

    DB = 512
    S = D // DB
    HB = 512
    J = H // HB

    W2p = jnp.zeros((H, _EPAD), jnp.float32).at[:, :E].set(W2)
    W2pb = W2p.astype(jnp.bfloat16)
    b2p = jnp.full((1, _EPAD), -1e30, jnp.float32).at[0, :E].set(b2)
    b1r = b1.reshape(1, H)
    sp = jnp.asarray(input, jnp.int32).reshape(1)

    grid_spec = pltpu.PrefetchScalarGridSpec(
        num_scalar_prefetch=1,
        grid=(S + J,),
        in_specs=[
            pl.BlockSpec((T, DB), lambda s, sp: (0, jnp.minimum(s, S - 1))),
            pl.BlockSpec((D, HB),
                         lambda s, sp: (0, jnp.clip(s - S, 0, J - 1))),
            pl.BlockSpec((1, HB),
                         lambda s, sp: (0, jnp.clip(s - S, 0, J - 1))),
            pl.BlockSpec((HB, _EPAD),
                         lambda s, sp: (jnp.clip(s - S, 0, J - 1), 0)),
            pl.BlockSpec((1, _EPAD), lambda s, sp: (0, 0)),
            pl.BlockSpec(memory_space=pl.ANY),
        ],
        out_specs=pl.BlockSpec((1, ED), lambda s, sp: (0, 0)),
        scratch_shapes=[
            pltpu.VMEM((S, T, DB), jnp.bfloat16),
            pltpu.VMEM((T, _EPAD), jnp.float32),
            pltpu.VMEM((1, ED), jnp.float32),
            pltpu.SemaphoreType.DMA,
        ],
    )

    out = pl.pallas_call(
        functools.partial(_body, E, S, DB, HB),
        grid_spec=grid_spec,
        out_shape=jax.ShapeDtypeStruct((1, ED), jnp.float32),
        compiler_params=pltpu.CompilerParams(
            dimension_semantics=("arbitrary",),
        ),
    )(sp, predicate, W1, b1r, W2pb, b2p, expert_tables)
    return out.reshape(ED)


# speedup vs baseline: 1.5857x; 1.0090x over previous
import functools

import jax
import jax.numpy as jnp
from jax.experimental import pallas as pl
from jax.experimental.pallas import tpu as pltpu

_EPAD = 128


def _body(E, S, DB, HB, sp_ref, xc_ref, w1_ref, b1_ref, w2_ref, b2_ref,
          tab_ref, out_ref, xbf_ref, log_ref, row_ref, sem):
    s = pl.program_id(0)
    ns = pl.num_programs(0)

    @pl.when(s < S)
    def _():
        xbf_ref[s] = xc_ref[...].astype(jnp.bfloat16)

    @pl.when(s >= S)
    def _():
        j = s - S
        w1b = w1_ref[...].astype(jnp.bfloat16)
        pre = jnp.zeros((xbf_ref.shape[1], HB), jnp.float32)
        for k in range(S):
            pre = pre + jnp.dot(xbf_ref[k],
                                w1b[k * DB:(k + 1) * DB, :],
                                preferred_element_type=jnp.float32)
        h = jax.nn.gelu(pre + b1_ref[...])
        w2b = w2_ref[...].astype(jnp.bfloat16)
        w2pb = jnp.concatenate(
            [w2b, jnp.zeros((HB, _EPAD - w2b.shape[1]), jnp.bfloat16)], axis=1)
        plog = jnp.dot(h.astype(jnp.bfloat16), w2pb,
                       preferred_element_type=jnp.float32)

        @pl.when(j == 0)
        def _():
            b2p = jnp.concatenate(
                [b2_ref[...],
                 jnp.full((1, _EPAD - b2_ref.shape[1]), -1e30, jnp.float32)],
                axis=1)
            log_ref[...] = plog + b2p

        @pl.when(j != 0)
        def _():
            log_ref[...] = log_ref[...] + plog

        @pl.when(s == ns - 1)
        def _():
            lg = log_ref[...]
            m = jnp.max(lg)
            rows = jax.lax.broadcasted_iota(jnp.int32, lg.shape, 0)
            cols = jax.lax.broadcasted_iota(jnp.int32, lg.shape, 1)
            flat = rows * E + cols
            idx = jnp.min(jnp.where(lg == m, flat, jnp.int32(2**30)))
            copy = pltpu.make_async_copy(
                tab_ref.at[sp_ref[0], pl.ds(idx, 1), :], row_ref, sem)
            copy.start()
            copy.wait()
            out_ref[...] = row_ref[...]


def kernel(predicate, W1, b1, W2, b2, expert_tables, input):
    T, D = predicate.shape
    H = W1.shape[1]
    E = W2.shape[1]
    n_tab, ROWS, ED = expert_tables.shape

    DB = 512
    S = D // DB
    HB = 512
    J = H // HB

    b1r = b1.reshape(1, H)
    b2r = b2.reshape(1, E)
    sp = jnp.asarray(input, jnp.int32).reshape(1)

    grid_spec = pltpu.PrefetchScalarGridSpec(
        num_scalar_prefetch=1,
        grid=(S + J,),
        in_specs=[
            pl.BlockSpec((T, DB), lambda s, sp: (0, jnp.minimum(s, S - 1))),
            pl.BlockSpec((D, HB),
                         lambda s, sp: (0, jnp.clip(s - S, 0, J - 1))),
            pl.BlockSpec((1, HB),
                         lambda s, sp: (0, jnp.clip(s - S, 0, J - 1))),
            pl.BlockSpec((HB, E),
                         lambda s, sp: (jnp.clip(s - S, 0, J - 1), 0)),
            pl.BlockSpec((1, E), lambda s, sp: (0, 0)),
            pl.BlockSpec(memory_space=pl.ANY),
        ],
        out_specs=pl.BlockSpec((1, ED), lambda s, sp: (0, 0)),
        scratch_shapes=[
            pltpu.VMEM((S, T, DB), jnp.bfloat16),
            pltpu.VMEM((T, _EPAD), jnp.float32),
            pltpu.VMEM((1, ED), jnp.float32),
            pltpu.SemaphoreType.DMA,
        ],
    )

    out = pl.pallas_call(
        functools.partial(_body, E, S, DB, HB),
        grid_spec=grid_spec,
        out_shape=jax.ShapeDtypeStruct((1, ED), jnp.float32),
        compiler_params=pltpu.CompilerParams(
            dimension_semantics=("arbitrary",),
        ),
    )(sp, predicate, W1, b1r, W2, b2r, expert_tables)
    return out.reshape(ED)


# speedup vs baseline: 2.2697x; 1.4313x over previous
import functools

import jax
import jax.numpy as jnp
from jax.experimental import pallas as pl
from jax.experimental.pallas import tpu as pltpu

_EPAD = 128


def _body(E, S, DB, HB, sp_ref, xc_ref, w1_ref, b1_ref, w2_ref, b2_ref,
          out_ref, xbf_ref, log_ref):
    s = pl.program_id(0)
    ns = pl.num_programs(0)

    @pl.when(s < S)
    def _():
        xbf_ref[s] = xc_ref[...].astype(jnp.bfloat16)

    @pl.when(s >= S)
    def _():
        j = s - S
        w1b = w1_ref[...].astype(jnp.bfloat16)
        pre = jnp.zeros((xbf_ref.shape[1], HB), jnp.float32)
        for k in range(S):
            pre = pre + jnp.dot(xbf_ref[k],
                                w1b[k * DB:(k + 1) * DB, :],
                                preferred_element_type=jnp.float32)
        h = jax.nn.gelu(pre + b1_ref[...])
        w2b = w2_ref[...].astype(jnp.bfloat16)
        w2pb = jnp.concatenate(
            [w2b, jnp.zeros((HB, _EPAD - w2b.shape[1]), jnp.bfloat16)], axis=1)
        plog = jnp.dot(h.astype(jnp.bfloat16), w2pb,
                       preferred_element_type=jnp.float32)

        @pl.when(j == 0)
        def _():
            b2p = jnp.concatenate(
                [b2_ref[...],
                 jnp.full((1, _EPAD - b2_ref.shape[1]), -1e30, jnp.float32)],
                axis=1)
            log_ref[...] = plog + b2p

        @pl.when(j != 0)
        def _():
            log_ref[...] = log_ref[...] + plog

        @pl.when(s == ns - 1)
        def _():
            lg = log_ref[...]
            m = jnp.max(lg)
            rows = jax.lax.broadcasted_iota(jnp.int32, lg.shape, 0)
            cols = jax.lax.broadcasted_iota(jnp.int32, lg.shape, 1)
            flat = rows * E + cols
            idx = jnp.min(jnp.where(lg == m, flat, jnp.int32(2**30)))
            out_ref[...] = jnp.broadcast_to(idx, out_ref.shape)


def kernel(predicate, W1, b1, W2, b2, expert_tables, input):
    T, D = predicate.shape
    H = W1.shape[1]
    E = W2.shape[1]
    n_tab, ROWS, ED = expert_tables.shape

    DB = 512
    S = D // DB
    HB = 512
    J = H // HB

    b1r = b1.reshape(1, H)
    b2r = b2.reshape(1, E)
    sp = jnp.asarray(input, jnp.int32).reshape(1)

    grid_spec = pltpu.PrefetchScalarGridSpec(
        num_scalar_prefetch=1,
        grid=(S + J,),
        in_specs=[
            pl.BlockSpec((T, DB), lambda s, sp: (0, jnp.minimum(s, S - 1))),
            pl.BlockSpec((D, HB),
                         lambda s, sp: (0, jnp.clip(s - S, 0, J - 1))),
            pl.BlockSpec((1, HB),
                         lambda s, sp: (0, jnp.clip(s - S, 0, J - 1))),
            pl.BlockSpec((HB, E),
                         lambda s, sp: (jnp.clip(s - S, 0, J - 1), 0)),
            pl.BlockSpec((1, E), lambda s, sp: (0, 0)),
        ],
        out_specs=pl.BlockSpec((1, 128), lambda s, sp: (0, 0)),
        scratch_shapes=[
            pltpu.VMEM((S, T, DB), jnp.bfloat16),
            pltpu.VMEM((T, _EPAD), jnp.float32),
        ],
    )

    idx = pl.pallas_call(
        functools.partial(_body, E, S, DB, HB),
        grid_spec=grid_spec,
        out_shape=jax.ShapeDtypeStruct((1, 128), jnp.int32),
        compiler_params=pltpu.CompilerParams(
            dimension_semantics=("arbitrary",),
        ),
    )(sp, predicate, W1, b1r, W2, b2r)
    return jnp.take(expert_tables[input], idx[0, 0], axis=0)
